# baseline (speedup 1.0000x reference)
import functools

import jax
import jax.numpy as jnp
from jax import lax
from jax.experimental import pallas as pl
from jax.experimental.pallas import tpu as pltpu
from jax.experimental.pallas import tpu_sc as plsc

NMS_THR = 0.35
MIN_THR = 0.25
PRE_NMS = 4096
BLK = 512
NBLK = PRE_NMS // BLK
TOPK = 100
B, Q, C = 4, 5000, 91
N = B * Q
TW = 128


QT = 1000


def _score_kernel(ts_ref, logits_ref, boxes_ref, plogits_ref, pboxes_ref,
                  cand_ref, table_ref):
    b = pl.program_id(0)
    img_h = ts_ref[b, 0]
    img_w = ts_ref[b, 1]

    def head(lg, bx):
        prob = jax.nn.sigmoid(lg[0])
        scores = jnp.max(prob, axis=-1)
        labels = (jnp.argmax(prob, axis=-1) + 1).astype(jnp.float32)
        cx = bx[0, :, 0]; cy = bx[0, :, 1]; w = bx[0, :, 2]; h = bx[0, :, 3]
        boxes = jnp.stack([(cx - 0.5 * w) * img_w, (cy - 0.5 * h) * img_h,
                           (cx + 0.5 * w) * img_w, (cy + 0.5 * h) * img_h],
                          axis=-1)
        return scores, labels, boxes

    scores, labels, boxes = head(logits_ref[...], boxes_ref[...])
    pscores, plabels, pboxes = head(plogits_ref[...], pboxes_ref[...])
    cand_ref[0, 0, :] = jnp.where(scores >= MIN_THR, scores, -1.0)
    table_ref[...] = jnp.concatenate(
        [boxes, labels[:, None], pboxes, plabels[:, None],
         pscores[:, None], jnp.zeros((QT, TW - 11), jnp.float32)], axis=1)


def _sc_gather(table, idx):
    info = plsc.get_sparse_core_info()
    nc = info.num_cores
    bpw = PRE_NMS // (nc * info.num_subcores)

    def body(table_hbm, idx_hbm, out_hbm, idx_v, rows_v, sem):
        wid = lax.axis_index("s") * nc + lax.axis_index("c")
        base = wid * bpw
        pltpu.sync_copy(idx_hbm.at[pl.ds(base, bpw)], idx_v)
        pltpu.async_copy(table_hbm.at[idx_v], rows_v, sem).wait()
        pltpu.sync_copy(rows_v, out_hbm.at[pl.ds(base, bpw)])

    k = pl.kernel(
        body,
        mesh=plsc.VectorSubcoreMesh(core_axis_name="c", subcore_axis_name="s"),
        out_type=jax.ShapeDtypeStruct((PRE_NMS, TW), jnp.float32),
        scratch_types=[
            pltpu.VMEM((bpw,), jnp.int32),
            pltpu.VMEM((bpw, TW), jnp.float32),
            pltpu.SemaphoreType.DMA,
        ])
    return k(table, idx)


def _pair_mask(boxes_a, boxes_b):
    ax1 = boxes_a[:, 0]; ay1 = boxes_a[:, 1]; ax2 = boxes_a[:, 2]; ay2 = boxes_a[:, 3]
    bx1 = boxes_b[:, 0]; by1 = boxes_b[:, 1]; bx2 = boxes_b[:, 2]; by2 = boxes_b[:, 3]
    area_a = (ax2 - ax1) * (ay2 - ay1)
    area_b = (bx2 - bx1) * (by2 - by1)
    ltx = jnp.maximum(ax1[:, None], bx1[None, :])
    lty = jnp.maximum(ay1[:, None], by1[None, :])
    rbx = jnp.minimum(ax2[:, None], bx2[None, :])
    rby = jnp.minimum(ay2[:, None], by2[None, :])
    wx = jnp.maximum(rbx - ltx, 0.0)
    wy = jnp.maximum(rby - lty, 0.0)
    inter = wx * wy
    union = area_a[:, None] + area_b[None, :] - inter
    iou = inter / jnp.maximum(union, 1e-6)
    return (iou > NMS_THR).astype(jnp.float32)


def _nms_kernel(scores_ref, g_ref, ob_ref, ol_ref, os_ref,
                opb_ref, opl_ref, oms_ref):
    top_scores = scores_ref[0, :]
    g = g_ref[...]
    sel_boxes = g[:, 0:4]
    labf = g[:, 4]

    max_coord = jnp.max(sel_boxes)
    nms_boxes = sel_boxes + (labf * (max_coord + 1.0))[:, None]

    keep0 = (top_scores >= MIN_THR).astype(jnp.float32)

    tri = (lax.broadcasted_iota(jnp.int32, (BLK, BLK), 0)
           < lax.broadcasted_iota(jnp.int32, (BLK, BLK), 1)).astype(jnp.float32)

    keeps = []
    for b in range(NBLK):
        boxes_b = lax.slice_in_dim(nms_boxes, b * BLK, (b + 1) * BLK, axis=0)
        supp = jnp.zeros((1, BLK), jnp.float32)
        for a in range(b):
            boxes_a = lax.slice_in_dim(nms_boxes, a * BLK, (a + 1) * BLK, axis=0)
            m_ab = _pair_mask(boxes_a, boxes_b)
            supp = supp + jnp.dot(keeps[a], m_ab,
                                  preferred_element_type=jnp.float32)
        base = (lax.slice_in_dim(keep0, b * BLK, (b + 1) * BLK)[None, :]
                * (supp == 0.0).astype(jnp.float32))
        m_bb = _pair_mask(boxes_b, boxes_b) * tri

        def fix_cond(carry):
            kb, prev, it = carry
            return jnp.logical_and(jnp.any(kb != prev), it < BLK)

        def fix_body(carry):
            kb, prev, it = carry
            hit = jnp.dot(kb, m_bb, preferred_element_type=jnp.float32)
            kb_new = base * (hit == 0.0).astype(jnp.float32)
            return kb_new, kb, it + 1

        kb, _, _ = lax.while_loop(
            fix_cond, fix_body,
            (base, -jnp.ones((1, BLK), jnp.float32), jnp.int32(0)))
        keeps.append(kb)

    keep = jnp.concatenate(keeps, axis=1)[0]
    kept_scores = jnp.where(keep > 0.0, top_scores, -1.0)

    kf = keep
    nkeep = jnp.sum(kf)

    def prefix_sum(v):
        row = v[None, :]
        s = 1
        while s < PRE_NMS:
            row = row + jnp.concatenate(
                [jnp.zeros((1, s), jnp.float32), row[:, :-s]], axis=1)
            s *= 2
        return row[0]

    ck = prefix_sum(kf) - kf
    cn = prefix_sum(1.0 - kf) - (1.0 - kf)
    rank = jnp.where(kf > 0.0, ck, nkeep + cn)

    rr = lax.broadcasted_iota(jnp.int32, (TOPK, PRE_NMS), 0)
    onehot = (rank.astype(jnp.int32)[None, :] == rr).astype(jnp.float32)

    def gather4096(v):
        return jnp.sum(onehot * v[None, :], axis=1)

    final_scores = gather4096(kept_scores)
    det_scores = jnp.maximum(final_scores, 0.0)
    out_labels = gather4096(labf).astype(jnp.int32)
    ob = [gather4096(g[:, c]) for c in range(4)]
    opb = [gather4096(g[:, 5 + c]) for c in range(4)]
    out_plabels = gather4096(g[:, 9]).astype(jnp.int32)
    mot = gather4096(g[:, 10]) * det_scores

    ob_ref[...] = jnp.stack(ob, axis=-1)
    ol_ref[0, :] = out_labels
    os_ref[0, :] = det_scores
    opb_ref[...] = jnp.stack(opb, axis=-1)
    opl_ref[0, :] = out_plabels
    oms_ref[0, :] = mot


@jax.jit
def _run(pred_logits, pred_boxes, puppet_pred_logits, puppet_pred_boxes,
         target_sizes):
    ts = target_sizes.astype(jnp.float32)
    full = lambda s: pl.BlockSpec(s, lambda: tuple(0 for _ in s))
    nq = Q // QT
    sk = pl.pallas_call(
        _score_kernel,
        grid=(B, nq),
        in_specs=[
            pl.BlockSpec((B, 2), lambda b, q: (0, 0)),
            pl.BlockSpec((1, QT, C), lambda b, q: (b, q, 0)),
            pl.BlockSpec((1, QT, 4), lambda b, q: (b, q, 0)),
            pl.BlockSpec((1, QT, C), lambda b, q: (b, q, 0)),
            pl.BlockSpec((1, QT, 4), lambda b, q: (b, q, 0)),
        ],
        out_specs=[
            pl.BlockSpec((1, 1, QT), lambda b, q: (b * (Q // QT) + q, 0, 0)),
            pl.BlockSpec((QT, TW), lambda b, q: (b * (Q // QT) + q, 0)),
        ],
        out_shape=[
            jax.ShapeDtypeStruct((B * nq, 1, QT), jnp.float32),
            jax.ShapeDtypeStruct((N, TW), jnp.float32),
        ],
    )
    cand, table = sk(ts, pred_logits, pred_boxes,
                     puppet_pred_logits, puppet_pred_boxes)

    top_scores, top_idx = lax.top_k(cand.reshape(N), PRE_NMS)
    gathered = _sc_gather(table, top_idx)

    nk = pl.pallas_call(
        _nms_kernel,
        in_specs=[full((1, PRE_NMS)), full((PRE_NMS, TW))],
        out_specs=[
            full((TOPK, 4)), full((1, TOPK)), full((1, TOPK)),
            full((TOPK, 4)), full((1, TOPK)), full((1, TOPK)),
        ],
        out_shape=[
            jax.ShapeDtypeStruct((TOPK, 4), jnp.float32),
            jax.ShapeDtypeStruct((1, TOPK), jnp.int32),
            jax.ShapeDtypeStruct((1, TOPK), jnp.float32),
            jax.ShapeDtypeStruct((TOPK, 4), jnp.float32),
            jax.ShapeDtypeStruct((1, TOPK), jnp.int32),
            jax.ShapeDtypeStruct((1, TOPK), jnp.float32),
        ],
    )
    ob, ol, osc, opb, opl, oms = nk(top_scores[None, :], gathered)
    return (ob, ol[0], osc[0], opb, opl[0], oms[0])


def kernel(pred_logits, pred_boxes, puppet_pred_logits, puppet_pred_boxes,
           target_sizes, topk):
    del topk
    return _run(pred_logits, pred_boxes, puppet_pred_logits,
                puppet_pred_boxes, target_sizes)
